# Initial kernel scaffold; baseline (speedup 1.0000x reference)
import functools

import jax
import jax.numpy as jnp
from jax import lax
from jax.experimental import pallas as pl
from jax.experimental.pallas import tpu as pltpu
from jax.experimental.pallas import tpu_sc as plsc

N = 10000
E = 320000
DF = 128
DH = 16
NCLS = 10
NC, NS, L = 2, 16, 16
NW = NC * NS
NPAD = 10240
RPT = NPAD // NS
CHUNK = 128
EPW = E // NW
NCH = 80
EPWP = NCH * CHUNK

_mesh = plsc.VectorSubcoreMesh(core_axis_name="c", subcore_axis_name="s")


def _bcast_lane(vec, j):
    idx = jnp.full((L, 1), j, dtype=jnp.int32)
    dnums = lax.GatherDimensionNumbers(
        offset_dims=(), collapsed_slice_dims=(0,), start_index_map=(0,))
    return lax.gather(vec, idx, dnums, (1,),
                      mode=lax.GatherScatterMode.PROMISE_IN_BOUNDS)


@functools.partial(
    pl.kernel,
    out_type=jax.ShapeDtypeStruct((NC, NPAD), jnp.float32),
    mesh=_mesh,
    scratch_types=[
        pltpu.VMEM((NCH, CHUNK), jnp.int32),
        pltpu.VMEM((NCH, CHUNK), jnp.float32),
        pltpu.VMEM((CHUNK,), jnp.float32),
        pltpu.VMEM_SHARED((NPAD,), jnp.float32),
    ],
)
def _deg_kernel(col_hbm, ew_hbm, out_hbm, col_v, ew_v, zbuf, deg_sh):
    c = lax.axis_index("c")
    s = lax.axis_index("s")
    wid = c * NS + s

    @pl.loop(0, CHUNK, step=L)
    def _(i):
        zbuf[pl.ds(i, L)] = jnp.zeros((L,), jnp.float32)

    for t in range(RPT // CHUNK):
        pltpu.sync_copy(zbuf, deg_sh.at[pl.ds(s * RPT + t * CHUNK, CHUNK)])
    pltpu.sync_copy(col_hbm.at[wid], col_v)
    pltpu.sync_copy(ew_hbm.at[wid], ew_v)
    plsc.subcore_barrier()

    @pl.loop(0, NCH)
    def _(k):
        pltpu.sync_copy(ew_v.at[k], deg_sh.at[col_v.at[k]], add=True)

    plsc.subcore_barrier()

    @pl.when(s == 0)
    def _():
        pltpu.sync_copy(deg_sh, out_hbm.at[c])


@functools.partial(
    pl.kernel,
    out_type=jax.ShapeDtypeStruct((NC, NPAD, DH), jnp.float32),
    mesh=_mesh,
    scratch_types=[
        pltpu.VMEM((NCH, CHUNK), jnp.int32),
        pltpu.VMEM((NCH, CHUNK), jnp.int32),
        pltpu.VMEM((NCH, CHUNK), jnp.float32),
        pltpu.VMEM((CHUNK, DH), jnp.float32),
        pltpu.VMEM_SHARED((NPAD, DH), jnp.float32),
        pltpu.VMEM_SHARED((NPAD, DH), jnp.float32),
    ],
)
def _msg_kernel(y_hbm, row_hbm, col_hbm, ew_hbm, out_hbm,
                row_v, col_v, ew_v, buf, y_sh, agg_sh):
    c = lax.axis_index("c")
    s = lax.axis_index("s")
    wid = c * NS + s

    pltpu.sync_copy(y_hbm.at[pl.ds(s * RPT, RPT)], y_sh.at[pl.ds(s * RPT, RPT)])

    @pl.loop(0, CHUNK)
    def _(i):
        buf.at[i][...] = jnp.zeros((L,), jnp.float32)

    for t in range(RPT // CHUNK):
        pltpu.sync_copy(buf, agg_sh.at[pl.ds(s * RPT + t * CHUNK, CHUNK)])
    pltpu.sync_copy(row_hbm.at[wid], row_v)
    pltpu.sync_copy(col_hbm.at[wid], col_v)
    pltpu.sync_copy(ew_hbm.at[wid], ew_v)
    plsc.subcore_barrier()

    @pl.loop(0, NCH)
    def _(k):
        pltpu.sync_copy(y_sh.at[row_v.at[k]], buf)

        @pl.loop(0, CHUNK, step=L)
        def _(g):
            ew16 = ew_v.at[k][pl.ds(g, L)]
            for j in range(L):
                w = _bcast_lane(ew16, j)
                buf.at[g + j][...] = buf.at[g + j][...] * w

        pltpu.sync_copy(buf, agg_sh.at[col_v.at[k]], add=True)

    plsc.subcore_barrier()
    pltpu.sync_copy(agg_sh.at[pl.ds(s * RPT, RPT)],
                    out_hbm.at[c, pl.ds(s * RPT, RPT)])


def _tc1_body(x_ref, w_ref, o_ref):
    o_ref[...] = jnp.dot(x_ref[...], w_ref[...],
                         preferred_element_type=jnp.float32)


_tc1 = pl.pallas_call(
    _tc1_body, out_shape=jax.ShapeDtypeStruct((N, DH), jnp.float32))


def _tc2_body(deg_ref, xw_ref, y_ref, dis_ref):
    deg = deg_ref[0, :] + deg_ref[1, :] + 1.0
    dis = lax.rsqrt(deg)
    dis_ref[...] = dis
    y_ref[:N, :] = xw_ref[...] * dis[:N, None]
    y_ref[N:, :] = jnp.zeros((NPAD - N, DH), jnp.float32)


_tc2 = pl.pallas_call(
    _tc2_body,
    out_shape=(jax.ShapeDtypeStruct((NPAD, DH), jnp.float32),
               jax.ShapeDtypeStruct((NPAD,), jnp.float32)))


def _tc3_body(agg_ref, y_ref, dis_ref, b1_ref, w2_ref, y2_ref):
    z = agg_ref[0] + agg_ref[1] + y_ref[...]
    h = jnp.maximum(z * dis_ref[...][:, None] + b1_ref[...], 0.0)
    xw2 = jnp.dot(h, w2_ref[...], preferred_element_type=jnp.float32)
    y2_ref[...] = xw2 * dis_ref[...][:, None]


_tc3 = pl.pallas_call(
    _tc3_body, out_shape=jax.ShapeDtypeStruct((NPAD, DH), jnp.float32))


def _tc4_body(agg_ref, y2_ref, dis_ref, b2_ref, o_ref):
    z = agg_ref[0] + agg_ref[1] + y2_ref[...]
    logits = (z * dis_ref[...][:, None] + b2_ref[...])[:N, :NCLS]
    m = jnp.max(logits, axis=1, keepdims=True)
    lse = jnp.log(jnp.sum(jnp.exp(logits - m), axis=1, keepdims=True)) + m
    o_ref[...] = logits - lse


_tc4 = pl.pallas_call(
    _tc4_body, out_shape=jax.ShapeDtypeStruct((N, NCLS), jnp.float32))


def _pad_edges(a, fill):
    a = a.reshape(NW, EPW)
    pad = jnp.full((NW, EPWP - EPW), fill, dtype=a.dtype)
    return jnp.concatenate([a, pad], axis=1).reshape(NW, NCH, CHUNK)


def kernel(x, edge_index, edge_weight, W1, b1, W2, b2):
    rowp = _pad_edges(edge_index[0].astype(jnp.int32), 0)
    colp = _pad_edges(edge_index[1].astype(jnp.int32), 0)
    ewp = _pad_edges(edge_weight, 0.0)
    W2p = jnp.pad(W2, ((0, 0), (0, DH - NCLS)))
    b2p = jnp.pad(b2, (0, DH - NCLS))

    deg2 = _deg_kernel(colp, ewp)
    xw1 = _tc1(x, W1)
    y1, dis = _tc2(deg2, xw1)
    agg1 = _msg_kernel(y1, rowp, colp, ewp)
    y2 = _tc3(agg1, y1, dis, b1, W2p)
    agg2 = _msg_kernel(y2, rowp, colp, ewp)
    return _tc4(agg2, y2, dis, b2p)


# speedup vs baseline: 25.4374x; 25.4374x over previous
import functools

import jax
import jax.numpy as jnp
from jax import lax
from jax.experimental import pallas as pl
from jax.experimental.pallas import tpu as pltpu
from jax.experimental.pallas import tpu_sc as plsc

N = 10000
E = 320000
DF = 128
DH = 16
NCLS = 10
NC, NS, L = 2, 16, 16
NW = NC * NS
NPAD = 10240
RPT = NPAD // NS
CHUNK = 128
EPW = E // NW
NCH = 80
EPWP = NCH * CHUNK

_mesh = plsc.VectorSubcoreMesh(core_axis_name="c", subcore_axis_name="s")
_sc_params = pltpu.CompilerParams(use_tc_tiling_on_sc=False)


def _bcast_lane(vec, j):
    idx = jnp.full((L, 1), j, dtype=jnp.int32)
    dnums = lax.GatherDimensionNumbers(
        offset_dims=(), collapsed_slice_dims=(0,), start_index_map=(0,))
    return lax.gather(vec, idx, dnums, (1,),
                      mode=lax.GatherScatterMode.PROMISE_IN_BOUNDS)


def _zero_buf(buf):
    @pl.loop(0, CHUNK)
    def _(i):
        buf.at[i][...] = jnp.zeros((L,), jnp.float32)


def _export_slice(sh, out_hbm, c, s, buf):
    for t in range(RPT // CHUNK):
        off = s * RPT + t * CHUNK
        pltpu.sync_copy(sh.at[pl.ds(off, CHUNK)], buf)
        pltpu.sync_copy(buf, out_hbm.at[pl.ds(c * NPAD + off, CHUNK)])


@functools.partial(
    pl.kernel,
    out_type=jax.ShapeDtypeStruct((NC * NPAD, DH), jnp.float32),
    mesh=_mesh,
    scratch_types=[
        pltpu.VMEM((NCH, CHUNK), jnp.int32),
        pltpu.VMEM((NCH, CHUNK), jnp.float32),
        pltpu.VMEM((CHUNK, DH), jnp.float32),
        pltpu.VMEM_SHARED((NPAD, DH), jnp.float32),
    ],
    compiler_params=_sc_params,
)
def _deg_kernel(col_hbm, ew_hbm, out_hbm, col_v, ew_v, buf, deg_sh):
    c = lax.axis_index("c")
    s = lax.axis_index("s")
    wid = c * NS + s

    _zero_buf(buf)
    for t in range(RPT // CHUNK):
        pltpu.sync_copy(buf, deg_sh.at[pl.ds(s * RPT + t * CHUNK, CHUNK)])
    pltpu.sync_copy(col_hbm.at[wid], col_v)
    pltpu.sync_copy(ew_hbm.at[wid], ew_v)
    plsc.subcore_barrier()

    @pl.loop(0, NCH)
    def _(k):
        @pl.loop(0, CHUNK, step=L)
        def _(g):
            ew16 = ew_v.at[k][pl.ds(g, L)]
            for j in range(L):
                buf.at[g + j][...] = buf.at[g + j][...] * 0.0 + _bcast_lane(ew16, j)

        pltpu.sync_copy(buf, deg_sh.at[col_v.at[k]], add=True)

    plsc.subcore_barrier()
    _export_slice(deg_sh, out_hbm, c, s, buf)


@functools.partial(
    pl.kernel,
    out_type=jax.ShapeDtypeStruct((NC * NPAD, DH), jnp.float32),
    mesh=_mesh,
    scratch_types=[
        pltpu.VMEM((NCH, CHUNK), jnp.int32),
        pltpu.VMEM((NCH, CHUNK), jnp.int32),
        pltpu.VMEM((NCH, CHUNK), jnp.float32),
        pltpu.VMEM((CHUNK, DH), jnp.float32),
        pltpu.VMEM_SHARED((NPAD, DH), jnp.float32),
    ],
    compiler_params=_sc_params,
)
def _msg_kernel(y_hbm, row_hbm, col_hbm, ew_hbm, out_hbm,
                row_v, col_v, ew_v, buf, agg_sh):
    c = lax.axis_index("c")
    s = lax.axis_index("s")
    wid = c * NS + s

    _zero_buf(buf)
    for t in range(RPT // CHUNK):
        pltpu.sync_copy(buf, agg_sh.at[pl.ds(s * RPT + t * CHUNK, CHUNK)])
    pltpu.sync_copy(row_hbm.at[wid], row_v)
    pltpu.sync_copy(col_hbm.at[wid], col_v)
    pltpu.sync_copy(ew_hbm.at[wid], ew_v)
    plsc.subcore_barrier()

    @pl.loop(0, NCH)
    def _(k):
        pltpu.sync_copy(y_hbm.at[row_v.at[k]], buf)

        @pl.loop(0, CHUNK, step=L)
        def _(g):
            ew16 = ew_v.at[k][pl.ds(g, L)]
            for j in range(L):
                w = _bcast_lane(ew16, j)
                buf.at[g + j][...] = buf.at[g + j][...] * w

        pltpu.sync_copy(buf, agg_sh.at[col_v.at[k]], add=True)

    plsc.subcore_barrier()
    _export_slice(agg_sh, out_hbm, c, s, buf)


def _tc1_body(x_ref, w_ref, o_ref):
    o_ref[...] = jnp.dot(x_ref[...], w_ref[...],
                         preferred_element_type=jnp.float32)


_tc1 = pl.pallas_call(
    _tc1_body, out_shape=jax.ShapeDtypeStruct((N, DH), jnp.float32))


def _tc2_body(deg_ref, xw_ref, y_ref, dis_ref):
    deg = deg_ref[:NPAD] + deg_ref[NPAD:]
    dis = lax.rsqrt(deg[:, 0:1] + 1.0)
    dis_ref[...] = dis
    y_ref[:N, :] = xw_ref[...] * dis[:N]
    y_ref[N:, :] = jnp.zeros((NPAD - N, DH), jnp.float32)


_tc2 = pl.pallas_call(
    _tc2_body,
    out_shape=(jax.ShapeDtypeStruct((NPAD, DH), jnp.float32),
               jax.ShapeDtypeStruct((NPAD, 1), jnp.float32)))


def _tc3_body(agg_ref, y_ref, dis_ref, b1_ref, w2_ref, y2_ref):
    z = agg_ref[:NPAD] + agg_ref[NPAD:] + y_ref[...]
    h = jnp.maximum(z * dis_ref[...] + b1_ref[...], 0.0)
    xw2 = jnp.dot(h, w2_ref[...], preferred_element_type=jnp.float32)
    y2_ref[...] = xw2 * dis_ref[...]


_tc3 = pl.pallas_call(
    _tc3_body, out_shape=jax.ShapeDtypeStruct((NPAD, DH), jnp.float32))


def _tc4_body(agg_ref, y2_ref, dis_ref, b2_ref, o_ref):
    z = agg_ref[:NPAD] + agg_ref[NPAD:] + y2_ref[...]
    logits = (z * dis_ref[...] + b2_ref[...])[:N, :NCLS]
    m = jnp.max(logits, axis=1, keepdims=True)
    lse = jnp.log(jnp.sum(jnp.exp(logits - m), axis=1, keepdims=True)) + m
    o_ref[...] = logits - lse


_tc4 = pl.pallas_call(
    _tc4_body, out_shape=jax.ShapeDtypeStruct((N, NCLS), jnp.float32))


def _pad_edges(a, fill):
    a = a.reshape(NW, EPW)
    pad = jnp.full((NW, EPWP - EPW), fill, dtype=a.dtype)
    return jnp.concatenate([a, pad], axis=1).reshape(NW, NCH, CHUNK)


def kernel(x, edge_index, edge_weight, W1, b1, W2, b2):
    rowp = _pad_edges(edge_index[0].astype(jnp.int32), 0)
    colp = _pad_edges(edge_index[1].astype(jnp.int32), 0)
    ewp = _pad_edges(edge_weight, 0.0)
    W2p = jnp.pad(W2, ((0, 0), (0, DH - NCLS)))
    b2p = jnp.pad(b2, (0, DH - NCLS))

    deg2 = _deg_kernel(colp, ewp)
    xw1 = _tc1(x, W1)
    y1, dis = _tc2(deg2, xw1)
    agg1 = _msg_kernel(y1, rowp, colp, ewp)
    y2 = _tc3(agg1, y1, dis, b1, W2p)
    agg2 = _msg_kernel(y2, rowp, colp, ewp)
    return _tc4(agg2, y2, dis, b2p)


# speedup vs baseline: 37.7944x; 1.4858x over previous
import functools

import jax
import jax.numpy as jnp
from jax import lax
from jax.experimental import pallas as pl
from jax.experimental.pallas import tpu as pltpu
from jax.experimental.pallas import tpu_sc as plsc

N = 10000
E = 320000
DF = 128
DH = 16
NCLS = 10
NC, NS, L = 2, 16, 16
NW = NC * NS
NPAD = 10240
RPT = NPAD // NS
CHUNK = 128
EPW = E // NW
NCH = 80
EPWP = NCH * CHUNK
NBUF = 4
NROUND = NCH // NBUF

_mesh = plsc.VectorSubcoreMesh(core_axis_name="c", subcore_axis_name="s")
_sc_params = pltpu.CompilerParams(use_tc_tiling_on_sc=False)


def _bcast_lane(vec, j):
    idx = jnp.full((L, 1), j, dtype=jnp.int32)
    dnums = lax.GatherDimensionNumbers(
        offset_dims=(), collapsed_slice_dims=(0,), start_index_map=(0,))
    return lax.gather(vec, idx, dnums, (1,),
                      mode=lax.GatherScatterMode.PROMISE_IN_BOUNDS)


def _zero_buf(buf):
    @pl.loop(0, CHUNK)
    def _(i):
        buf.at[i][...] = jnp.zeros((L,), jnp.float32)


def _export_slice(sh, out_hbm, c, s, buf):
    for t in range(RPT // CHUNK):
        off = s * RPT + t * CHUNK
        pltpu.sync_copy(sh.at[pl.ds(off, CHUNK)], buf)
        pltpu.sync_copy(buf, out_hbm.at[pl.ds(c * NPAD + off, CHUNK)])


@functools.partial(
    pl.kernel,
    out_type=jax.ShapeDtypeStruct((NC * NPAD, DH), jnp.float32),
    mesh=_mesh,
    scratch_types=[
        pltpu.VMEM((NCH, CHUNK), jnp.int32),
        pltpu.VMEM((NCH, CHUNK), jnp.float32),
    ] + [pltpu.VMEM((CHUNK, DH), jnp.float32)] * NBUF + [
        pltpu.VMEM_SHARED((NPAD, DH), jnp.float32),
    ] + [pltpu.SemaphoreType.DMA] * NBUF,
    compiler_params=_sc_params,
)
def _deg_kernel(col_hbm, ew_hbm, out_hbm, col_v, ew_v, *rest):
    sbufs = rest[:NBUF]
    deg_sh = rest[NBUF]
    ssems = rest[NBUF + 1:]
    c = lax.axis_index("c")
    s = lax.axis_index("s")
    wid = c * NS + s

    _zero_buf(sbufs[0])
    for t in range(RPT // CHUNK):
        pltpu.sync_copy(sbufs[0], deg_sh.at[pl.ds(s * RPT + t * CHUNK, CHUNK)])
    pltpu.sync_copy(col_hbm.at[wid], col_v)
    pltpu.sync_copy(ew_hbm.at[wid], ew_v)
    plsc.subcore_barrier()

    @pl.loop(0, NROUND)
    def _(m):
        for b in range(NBUF):
            k = m * NBUF + b

            @pl.when(m > 0)
            def _():
                pltpu.make_async_copy(
                    sbufs[b], deg_sh.at[col_v.at[k - NBUF]], ssems[b]).wait()

            @pl.loop(0, CHUNK, step=L)
            def _(g):
                ew16 = ew_v.at[k][pl.ds(g, L)]
                for j in range(L):
                    sbufs[b].at[g + j][...] = (
                        sbufs[b].at[g + j][...] * 0.0 + _bcast_lane(ew16, j))

            pltpu.async_copy(sbufs[b], deg_sh.at[col_v.at[k]], ssems[b],
                             add=True)

    for b in range(NBUF):
        pltpu.make_async_copy(
            sbufs[b], deg_sh.at[col_v.at[NCH - NBUF + b]], ssems[b]).wait()
    plsc.subcore_barrier()
    _export_slice(deg_sh, out_hbm, c, s, sbufs[0])


@functools.partial(
    pl.kernel,
    out_type=jax.ShapeDtypeStruct((NC * NPAD, DH), jnp.float32),
    mesh=_mesh,
    scratch_types=[
        pltpu.VMEM((NCH, CHUNK), jnp.int32),
        pltpu.VMEM((NCH, CHUNK), jnp.int32),
        pltpu.VMEM((NCH, CHUNK), jnp.float32),
    ] + [pltpu.VMEM((CHUNK, DH), jnp.float32)] * (2 * NBUF) + [
        pltpu.VMEM_SHARED((NPAD, DH), jnp.float32),
    ] + [pltpu.SemaphoreType.DMA] * (2 * NBUF),
    compiler_params=_sc_params,
)
def _msg_kernel(y_hbm, row_hbm, col_hbm, ew_hbm, out_hbm,
                row_v, col_v, ew_v, *rest):
    gbufs = rest[:NBUF]
    sbufs = rest[NBUF:2 * NBUF]
    agg_sh = rest[2 * NBUF]
    gsems = rest[2 * NBUF + 1:3 * NBUF + 1]
    ssems = rest[3 * NBUF + 1:]
    c = lax.axis_index("c")
    s = lax.axis_index("s")
    wid = c * NS + s

    _zero_buf(sbufs[0])
    for t in range(RPT // CHUNK):
        pltpu.sync_copy(sbufs[0], agg_sh.at[pl.ds(s * RPT + t * CHUNK, CHUNK)])
    pltpu.sync_copy(row_hbm.at[wid], row_v)
    pltpu.sync_copy(col_hbm.at[wid], col_v)
    pltpu.sync_copy(ew_hbm.at[wid], ew_v)
    plsc.subcore_barrier()

    for b in range(NBUF):
        pltpu.async_copy(y_hbm.at[row_v.at[b]], gbufs[b], gsems[b])

    @pl.loop(0, NROUND)
    def _(m):
        for b in range(NBUF):
            k = m * NBUF + b

            @pl.when(m > 0)
            def _():
                pltpu.make_async_copy(
                    sbufs[b], agg_sh.at[col_v.at[k - NBUF]], ssems[b]).wait()

            pltpu.make_async_copy(
                y_hbm.at[row_v.at[k]], gbufs[b], gsems[b]).wait()

            @pl.loop(0, CHUNK, step=L)
            def _(g):
                ew16 = ew_v.at[k][pl.ds(g, L)]
                for j in range(L):
                    w = _bcast_lane(ew16, j)
                    sbufs[b].at[g + j][...] = gbufs[b].at[g + j][...] * w

            @pl.when(m < NROUND - 1)
            def _():
                pltpu.async_copy(
                    y_hbm.at[row_v.at[k + NBUF]], gbufs[b], gsems[b])

            pltpu.async_copy(sbufs[b], agg_sh.at[col_v.at[k]], ssems[b],
                             add=True)

    for b in range(NBUF):
        pltpu.make_async_copy(
            sbufs[b], agg_sh.at[col_v.at[NCH - NBUF + b]], ssems[b]).wait()
    plsc.subcore_barrier()
    _export_slice(agg_sh, out_hbm, c, s, sbufs[0])


def _tc1_body(x_ref, w_ref, o_ref):
    o_ref[...] = jnp.dot(x_ref[...], w_ref[...],
                         preferred_element_type=jnp.float32)


_tc1 = pl.pallas_call(
    _tc1_body, out_shape=jax.ShapeDtypeStruct((N, DH), jnp.float32))


def _tc2_body(deg_ref, xw_ref, y_ref, dis_ref):
    deg = deg_ref[:NPAD] + deg_ref[NPAD:]
    dis = lax.rsqrt(deg[:, 0:1] + 1.0)
    dis_ref[...] = dis
    y_ref[:N, :] = xw_ref[...] * dis[:N]
    y_ref[N:, :] = jnp.zeros((NPAD - N, DH), jnp.float32)


_tc2 = pl.pallas_call(
    _tc2_body,
    out_shape=(jax.ShapeDtypeStruct((NPAD, DH), jnp.float32),
               jax.ShapeDtypeStruct((NPAD, 1), jnp.float32)))


def _tc3_body(agg_ref, y_ref, dis_ref, b1_ref, w2_ref, y2_ref):
    z = agg_ref[:NPAD] + agg_ref[NPAD:] + y_ref[...]
    h = jnp.maximum(z * dis_ref[...] + b1_ref[...], 0.0)
    xw2 = jnp.dot(h, w2_ref[...], preferred_element_type=jnp.float32)
    y2_ref[...] = xw2 * dis_ref[...]


_tc3 = pl.pallas_call(
    _tc3_body, out_shape=jax.ShapeDtypeStruct((NPAD, DH), jnp.float32))


def _tc4_body(agg_ref, y2_ref, dis_ref, b2_ref, o_ref):
    z = agg_ref[:NPAD] + agg_ref[NPAD:] + y2_ref[...]
    logits = (z * dis_ref[...] + b2_ref[...])[:N, :NCLS]
    m = jnp.max(logits, axis=1, keepdims=True)
    lse = jnp.log(jnp.sum(jnp.exp(logits - m), axis=1, keepdims=True)) + m
    o_ref[...] = logits - lse


_tc4 = pl.pallas_call(
    _tc4_body, out_shape=jax.ShapeDtypeStruct((N, NCLS), jnp.float32))


def _pad_edges(a, fill):
    a = a.reshape(NW, EPW)
    pad = jnp.full((NW, EPWP - EPW), fill, dtype=a.dtype)
    return jnp.concatenate([a, pad], axis=1).reshape(NW, NCH, CHUNK)


def kernel(x, edge_index, edge_weight, W1, b1, W2, b2):
    rowp = _pad_edges(edge_index[0].astype(jnp.int32), 0)
    colp = _pad_edges(edge_index[1].astype(jnp.int32), 0)
    ewp = _pad_edges(edge_weight, 0.0)
    W2p = jnp.pad(W2, ((0, 0), (0, DH - NCLS)))
    b2p = jnp.pad(b2, (0, DH - NCLS))

    deg2 = _deg_kernel(colp, ewp)
    xw1 = _tc1(x, W1)
    y1, dis = _tc2(deg2, xw1)
    agg1 = _msg_kernel(y1, rowp, colp, ewp)
    y2 = _tc3(agg1, y1, dis, b1, W2p)
    agg2 = _msg_kernel(y2, rowp, colp, ewp)
    return _tc4(agg2, y2, dis, b2p)


# speedup vs baseline: 39.6714x; 1.0497x over previous
import functools

import jax
import jax.numpy as jnp
from jax import lax
from jax.experimental import pallas as pl
from jax.experimental.pallas import tpu as pltpu
from jax.experimental.pallas import tpu_sc as plsc

N = 10000
E = 320000
DF = 128
DH = 16
NCLS = 10
NC, NS, L = 2, 16, 16
NW = NC * NS
NPAD = 10240
RPT = NPAD // NS
CHUNK = 128
EPW = E // NW
NCH = 80
EPWP = NCH * CHUNK
NBUF = 8
NROUND = NCH // NBUF

_mesh = plsc.VectorSubcoreMesh(core_axis_name="c", subcore_axis_name="s")
_sc_params = pltpu.CompilerParams(use_tc_tiling_on_sc=False)


def _bcast_lane(vec, j):
    idx = jnp.full((L, 1), j, dtype=jnp.int32)
    dnums = lax.GatherDimensionNumbers(
        offset_dims=(), collapsed_slice_dims=(0,), start_index_map=(0,))
    return lax.gather(vec, idx, dnums, (1,),
                      mode=lax.GatherScatterMode.PROMISE_IN_BOUNDS)


def _zero_buf(buf):
    @pl.loop(0, CHUNK)
    def _(i):
        buf.at[i][...] = jnp.zeros((L,), jnp.float32)


def _export_slice(sh, out_hbm, c, s, buf):
    for t in range(RPT // CHUNK):
        off = s * RPT + t * CHUNK
        pltpu.sync_copy(sh.at[pl.ds(off, CHUNK)], buf)
        pltpu.sync_copy(buf, out_hbm.at[pl.ds(c * NPAD + off, CHUNK)])


@functools.partial(
    pl.kernel,
    out_type=jax.ShapeDtypeStruct((NC * NPAD, DH), jnp.float32),
    mesh=_mesh,
    scratch_types=[
        pltpu.VMEM((NCH, CHUNK), jnp.int32),
        pltpu.VMEM((NCH, CHUNK), jnp.float32),
    ] + [pltpu.VMEM((CHUNK, DH), jnp.float32)] * NBUF + [
        pltpu.VMEM_SHARED((NPAD, DH), jnp.float32),
    ] + [pltpu.SemaphoreType.DMA] * NBUF,
    compiler_params=_sc_params,
)
def _deg_kernel(col_hbm, ew_hbm, out_hbm, col_v, ew_v, *rest):
    sbufs = rest[:NBUF]
    deg_sh = rest[NBUF]
    ssems = rest[NBUF + 1:]
    c = lax.axis_index("c")
    s = lax.axis_index("s")
    wid = c * NS + s

    _zero_buf(sbufs[0])
    for t in range(RPT // CHUNK):
        pltpu.sync_copy(sbufs[0], deg_sh.at[pl.ds(s * RPT + t * CHUNK, CHUNK)])
    pltpu.sync_copy(col_hbm.at[wid], col_v)
    pltpu.sync_copy(ew_hbm.at[wid], ew_v)
    plsc.subcore_barrier()

    @pl.loop(0, NROUND)
    def _(m):
        for b in range(NBUF):
            k = m * NBUF + b

            @pl.when(m > 0)
            def _():
                pltpu.make_async_copy(
                    sbufs[b], deg_sh.at[col_v.at[k - NBUF]], ssems[b]).wait()

            @pl.loop(0, CHUNK, step=L)
            def _(g):
                ew16 = ew_v.at[k][pl.ds(g, L)]
                for j in range(L):
                    sbufs[b].at[g + j][...] = (
                        sbufs[b].at[g + j][...] * 0.0 + _bcast_lane(ew16, j))

            pltpu.async_copy(sbufs[b], deg_sh.at[col_v.at[k]], ssems[b],
                             add=True)

    for b in range(NBUF):
        pltpu.make_async_copy(
            sbufs[b], deg_sh.at[col_v.at[NCH - NBUF + b]], ssems[b]).wait()
    plsc.subcore_barrier()
    _export_slice(deg_sh, out_hbm, c, s, sbufs[0])


@functools.partial(
    pl.kernel,
    out_type=jax.ShapeDtypeStruct((NC * NPAD, DH), jnp.float32),
    mesh=_mesh,
    scratch_types=[
        pltpu.VMEM((NCH, CHUNK), jnp.int32),
        pltpu.VMEM((NCH, CHUNK), jnp.int32),
        pltpu.VMEM((NCH, CHUNK), jnp.float32),
    ] + [pltpu.VMEM((CHUNK, DH), jnp.float32)] * (2 * NBUF) + [
        pltpu.VMEM_SHARED((NPAD, DH), jnp.float32),
    ] + [pltpu.SemaphoreType.DMA] * (2 * NBUF),
    compiler_params=_sc_params,
)
def _msg_kernel(y_hbm, row_hbm, col_hbm, ew_hbm, out_hbm,
                row_v, col_v, ew_v, *rest):
    gbufs = rest[:NBUF]
    sbufs = rest[NBUF:2 * NBUF]
    agg_sh = rest[2 * NBUF]
    gsems = rest[2 * NBUF + 1:3 * NBUF + 1]
    ssems = rest[3 * NBUF + 1:]
    c = lax.axis_index("c")
    s = lax.axis_index("s")
    wid = c * NS + s

    _zero_buf(sbufs[0])
    for t in range(RPT // CHUNK):
        pltpu.sync_copy(sbufs[0], agg_sh.at[pl.ds(s * RPT + t * CHUNK, CHUNK)])
    pltpu.sync_copy(row_hbm.at[wid], row_v)
    pltpu.sync_copy(col_hbm.at[wid], col_v)
    pltpu.sync_copy(ew_hbm.at[wid], ew_v)
    plsc.subcore_barrier()

    for b in range(NBUF):
        pltpu.async_copy(y_hbm.at[row_v.at[b]], gbufs[b], gsems[b])

    @pl.loop(0, NROUND)
    def _(m):
        for b in range(NBUF):
            k = m * NBUF + b

            @pl.when(m > 0)
            def _():
                pltpu.make_async_copy(
                    sbufs[b], agg_sh.at[col_v.at[k - NBUF]], ssems[b]).wait()

            pltpu.make_async_copy(
                y_hbm.at[row_v.at[k]], gbufs[b], gsems[b]).wait()

            @pl.loop(0, CHUNK, step=L)
            def _(g):
                ew16 = ew_v.at[k][pl.ds(g, L)]
                for j in range(L):
                    w = _bcast_lane(ew16, j)
                    sbufs[b].at[g + j][...] = gbufs[b].at[g + j][...] * w

            @pl.when(m < NROUND - 1)
            def _():
                pltpu.async_copy(
                    y_hbm.at[row_v.at[k + NBUF]], gbufs[b], gsems[b])

            pltpu.async_copy(sbufs[b], agg_sh.at[col_v.at[k]], ssems[b],
                             add=True)

    for b in range(NBUF):
        pltpu.make_async_copy(
            sbufs[b], agg_sh.at[col_v.at[NCH - NBUF + b]], ssems[b]).wait()
    plsc.subcore_barrier()
    _export_slice(agg_sh, out_hbm, c, s, sbufs[0])


def _tc1_body(x_ref, w_ref, o_ref):
    o_ref[...] = jnp.dot(x_ref[...], w_ref[...],
                         preferred_element_type=jnp.float32)


_tc1 = pl.pallas_call(
    _tc1_body, out_shape=jax.ShapeDtypeStruct((N, DH), jnp.float32))


def _tc2_body(deg_ref, xw_ref, y_ref, dis_ref):
    deg = deg_ref[:NPAD] + deg_ref[NPAD:]
    dis = lax.rsqrt(deg[:, 0:1] + 1.0)
    dis_ref[...] = dis
    y_ref[:N, :] = xw_ref[...] * dis[:N]
    y_ref[N:, :] = jnp.zeros((NPAD - N, DH), jnp.float32)


_tc2 = pl.pallas_call(
    _tc2_body,
    out_shape=(jax.ShapeDtypeStruct((NPAD, DH), jnp.float32),
               jax.ShapeDtypeStruct((NPAD, 1), jnp.float32)))


def _tc3_body(agg_ref, y_ref, dis_ref, b1_ref, w2_ref, y2_ref):
    z = agg_ref[:NPAD] + agg_ref[NPAD:] + y_ref[...]
    h = jnp.maximum(z * dis_ref[...] + b1_ref[...], 0.0)
    xw2 = jnp.dot(h, w2_ref[...], preferred_element_type=jnp.float32)
    y2_ref[...] = xw2 * dis_ref[...]


_tc3 = pl.pallas_call(
    _tc3_body, out_shape=jax.ShapeDtypeStruct((NPAD, DH), jnp.float32))


def _tc4_body(agg_ref, y2_ref, dis_ref, b2_ref, o_ref):
    z = agg_ref[:NPAD] + agg_ref[NPAD:] + y2_ref[...]
    logits = (z * dis_ref[...] + b2_ref[...])[:N, :NCLS]
    m = jnp.max(logits, axis=1, keepdims=True)
    lse = jnp.log(jnp.sum(jnp.exp(logits - m), axis=1, keepdims=True)) + m
    o_ref[...] = logits - lse


_tc4 = pl.pallas_call(
    _tc4_body, out_shape=jax.ShapeDtypeStruct((N, NCLS), jnp.float32))


def _pad_edges(a, fill):
    a = a.reshape(NW, EPW)
    pad = jnp.full((NW, EPWP - EPW), fill, dtype=a.dtype)
    return jnp.concatenate([a, pad], axis=1).reshape(NW, NCH, CHUNK)


def kernel(x, edge_index, edge_weight, W1, b1, W2, b2):
    rowp = _pad_edges(edge_index[0].astype(jnp.int32), 0)
    colp = _pad_edges(edge_index[1].astype(jnp.int32), 0)
    ewp = _pad_edges(edge_weight, 0.0)
    W2p = jnp.pad(W2, ((0, 0), (0, DH - NCLS)))
    b2p = jnp.pad(b2, (0, DH - NCLS))

    deg2 = _deg_kernel(colp, ewp)
    xw1 = _tc1(x, W1)
    y1, dis = _tc2(deg2, xw1)
    agg1 = _msg_kernel(y1, rowp, colp, ewp)
    y2 = _tc3(agg1, y1, dis, b1, W2p)
    agg2 = _msg_kernel(y2, rowp, colp, ewp)
    return _tc4(agg2, y2, dis, b2p)


# speedup vs baseline: 50.2690x; 1.2671x over previous
import functools

import jax
import jax.numpy as jnp
from jax import lax
from jax.experimental import pallas as pl
from jax.experimental.pallas import tpu as pltpu
from jax.experimental.pallas import tpu_sc as plsc

N = 10000
E = 320000
DF = 128
DH = 16
NCLS = 10
NC, NS, L = 2, 16, 16
NW = NC * NS
NPAD = 10240
RPT = NPAD // NS
CHUNK = 128
EPW = E // NW
NCH = 80
EPWP = NCH * CHUNK
NBUF = 8
NROUND = NCH // NBUF

_mesh = plsc.VectorSubcoreMesh(core_axis_name="c", subcore_axis_name="s")
_sc_params = pltpu.CompilerParams(use_tc_tiling_on_sc=False)


def _bcast_lane(vec, j):
    idx = jnp.full((L, 1), j, dtype=jnp.int32)
    dnums = lax.GatherDimensionNumbers(
        offset_dims=(), collapsed_slice_dims=(0,), start_index_map=(0,))
    return lax.gather(vec, idx, dnums, (1,),
                      mode=lax.GatherScatterMode.PROMISE_IN_BOUNDS)


def _zero_buf(buf):
    @pl.loop(0, CHUNK)
    def _(i):
        buf.at[i][...] = jnp.zeros((L,), jnp.float32)


def _export_slice(sh, out_hbm, c, s, buf):
    for t in range(RPT // CHUNK):
        off = s * RPT + t * CHUNK
        pltpu.sync_copy(sh.at[pl.ds(off, CHUNK)], buf)
        pltpu.sync_copy(buf, out_hbm.at[pl.ds(c * NPAD + off, CHUNK)])


@functools.partial(
    pl.kernel,
    out_type=jax.ShapeDtypeStruct((NC * NPAD, DH), jnp.float32),
    mesh=_mesh,
    scratch_types=[
        pltpu.VMEM((NCH, CHUNK), jnp.int32),
        pltpu.VMEM((NCH, CHUNK), jnp.float32),
    ] + [pltpu.VMEM((CHUNK, DH), jnp.float32)] * NBUF + [
        pltpu.VMEM_SHARED((NPAD, DH), jnp.float32),
    ] + [pltpu.SemaphoreType.DMA] * NBUF,
    compiler_params=_sc_params,
)
def _deg_kernel(col_hbm, ew_hbm, out_hbm, col_v, ew_v, *rest):
    sbufs = rest[:NBUF]
    deg_sh = rest[NBUF]
    ssems = rest[NBUF + 1:]
    c = lax.axis_index("c")
    s = lax.axis_index("s")
    wid = c * NS + s

    _zero_buf(sbufs[0])
    for t in range(RPT // CHUNK):
        pltpu.sync_copy(sbufs[0], deg_sh.at[pl.ds(s * RPT + t * CHUNK, CHUNK)])
    pltpu.sync_copy(col_hbm.at[wid], col_v)
    pltpu.sync_copy(ew_hbm.at[wid], ew_v)
    plsc.subcore_barrier()

    @pl.loop(0, NROUND)
    def _(m):
        for b in range(NBUF):
            k = m * NBUF + b

            @pl.when(m > 0)
            def _():
                pltpu.make_async_copy(
                    sbufs[b], deg_sh.at[col_v.at[k - NBUF]], ssems[b]).wait()

            @pl.loop(0, CHUNK, step=L)
            def _(g):
                ew16 = ew_v.at[k][pl.ds(g, L)]
                for j in range(L):
                    sbufs[b].at[g + j][...] = (
                        sbufs[b].at[g + j][...] * 0.0 + _bcast_lane(ew16, j))

            pltpu.async_copy(sbufs[b], deg_sh.at[col_v.at[k]], ssems[b],
                             add=True)

    for b in range(NBUF):
        pltpu.make_async_copy(
            sbufs[b], deg_sh.at[col_v.at[NCH - NBUF + b]], ssems[b]).wait()
    plsc.subcore_barrier()
    _export_slice(deg_sh, out_hbm, c, s, sbufs[0])


@functools.partial(
    pl.kernel,
    out_type=jax.ShapeDtypeStruct((NC * NPAD, DH), jnp.float32),
    mesh=_mesh,
    scratch_types=[
        pltpu.VMEM((NCH, CHUNK), jnp.int32),
        pltpu.VMEM((NCH, CHUNK), jnp.int32),
        pltpu.VMEM((NCH, CHUNK), jnp.float32),
    ] + [pltpu.VMEM((CHUNK, DH), jnp.float32)] * (2 * NBUF) + [
        pltpu.VMEM_SHARED((NPAD, DH), jnp.float32),
        pltpu.VMEM_SHARED((NPAD, DH), jnp.float32),
    ] + [pltpu.SemaphoreType.DMA] * (2 * NBUF),
    compiler_params=_sc_params,
)
def _msg_kernel(y_hbm, row_hbm, col_hbm, ew_hbm, out_hbm,
                row_v, col_v, ew_v, *rest):
    gbufs = rest[:NBUF]
    sbufs = rest[NBUF:2 * NBUF]
    agg_sh = rest[2 * NBUF]
    y_sh = rest[2 * NBUF + 1]
    gsems = rest[2 * NBUF + 2:3 * NBUF + 2]
    ssems = rest[3 * NBUF + 2:]
    c = lax.axis_index("c")
    s = lax.axis_index("s")
    wid = c * NS + s

    for t in range(RPT // CHUNK):
        sl = pl.ds(s * RPT + t * CHUNK, CHUNK)
        pltpu.sync_copy(y_hbm.at[sl], sbufs[0])
        pltpu.sync_copy(sbufs[0], y_sh.at[sl])
    _zero_buf(sbufs[0])
    for t in range(RPT // CHUNK):
        pltpu.sync_copy(sbufs[0], agg_sh.at[pl.ds(s * RPT + t * CHUNK, CHUNK)])
    pltpu.sync_copy(row_hbm.at[wid], row_v)
    pltpu.sync_copy(col_hbm.at[wid], col_v)
    pltpu.sync_copy(ew_hbm.at[wid], ew_v)
    plsc.subcore_barrier()

    for b in range(NBUF):
        pltpu.async_copy(y_sh.at[row_v.at[b]], gbufs[b], gsems[b])

    @pl.loop(0, NROUND)
    def _(m):
        for b in range(NBUF):
            k = m * NBUF + b

            @pl.when(m > 0)
            def _():
                pltpu.make_async_copy(
                    sbufs[b], agg_sh.at[col_v.at[k - NBUF]], ssems[b]).wait()

            pltpu.make_async_copy(
                y_sh.at[row_v.at[k]], gbufs[b], gsems[b]).wait()

            @pl.loop(0, CHUNK, step=L)
            def _(g):
                ew16 = ew_v.at[k][pl.ds(g, L)]
                for j in range(L):
                    w = _bcast_lane(ew16, j)
                    sbufs[b].at[g + j][...] = gbufs[b].at[g + j][...] * w

            @pl.when(m < NROUND - 1)
            def _():
                pltpu.async_copy(
                    y_sh.at[row_v.at[k + NBUF]], gbufs[b], gsems[b])

            pltpu.async_copy(sbufs[b], agg_sh.at[col_v.at[k]], ssems[b],
                             add=True)

    for b in range(NBUF):
        pltpu.make_async_copy(
            sbufs[b], agg_sh.at[col_v.at[NCH - NBUF + b]], ssems[b]).wait()
    plsc.subcore_barrier()
    _export_slice(agg_sh, out_hbm, c, s, sbufs[0])


def _tc1_body(x_ref, w_ref, o_ref):
    o_ref[...] = jnp.dot(x_ref[...], w_ref[...],
                         preferred_element_type=jnp.float32)


_tc1 = pl.pallas_call(
    _tc1_body, out_shape=jax.ShapeDtypeStruct((N, DH), jnp.float32))


def _tc2_body(deg_ref, xw_ref, y_ref, dis_ref):
    deg = deg_ref[:NPAD] + deg_ref[NPAD:]
    dis = lax.rsqrt(deg[:, 0:1] + 1.0)
    dis_ref[...] = dis
    y_ref[:N, :] = xw_ref[...] * dis[:N]
    y_ref[N:, :] = jnp.zeros((NPAD - N, DH), jnp.float32)


_tc2 = pl.pallas_call(
    _tc2_body,
    out_shape=(jax.ShapeDtypeStruct((NPAD, DH), jnp.float32),
               jax.ShapeDtypeStruct((NPAD, 1), jnp.float32)))


def _tc3_body(agg_ref, y_ref, dis_ref, b1_ref, w2_ref, y2_ref):
    z = agg_ref[:NPAD] + agg_ref[NPAD:] + y_ref[...]
    h = jnp.maximum(z * dis_ref[...] + b1_ref[...], 0.0)
    xw2 = jnp.dot(h, w2_ref[...], preferred_element_type=jnp.float32)
    y2_ref[...] = xw2 * dis_ref[...]


_tc3 = pl.pallas_call(
    _tc3_body, out_shape=jax.ShapeDtypeStruct((NPAD, DH), jnp.float32))


def _tc4_body(agg_ref, y2_ref, dis_ref, b2_ref, o_ref):
    z = agg_ref[:NPAD] + agg_ref[NPAD:] + y2_ref[...]
    logits = (z * dis_ref[...] + b2_ref[...])[:N, :NCLS]
    m = jnp.max(logits, axis=1, keepdims=True)
    lse = jnp.log(jnp.sum(jnp.exp(logits - m), axis=1, keepdims=True)) + m
    o_ref[...] = logits - lse


_tc4 = pl.pallas_call(
    _tc4_body, out_shape=jax.ShapeDtypeStruct((N, NCLS), jnp.float32))


def _pad_edges(a, fill):
    a = a.reshape(NW, EPW)
    pad = jnp.full((NW, EPWP - EPW), fill, dtype=a.dtype)
    return jnp.concatenate([a, pad], axis=1).reshape(NW, NCH, CHUNK)


def kernel(x, edge_index, edge_weight, W1, b1, W2, b2):
    rowp = _pad_edges(edge_index[0].astype(jnp.int32), 0)
    colp = _pad_edges(edge_index[1].astype(jnp.int32), 0)
    ewp = _pad_edges(edge_weight, 0.0)
    W2p = jnp.pad(W2, ((0, 0), (0, DH - NCLS)))
    b2p = jnp.pad(b2, (0, DH - NCLS))

    deg2 = _deg_kernel(colp, ewp)
    xw1 = _tc1(x, W1)
    y1, dis = _tc2(deg2, xw1)
    agg1 = _msg_kernel(y1, rowp, colp, ewp)
    y2 = _tc3(agg1, y1, dis, b1, W2p)
    agg2 = _msg_kernel(y2, rowp, colp, ewp)
    return _tc4(agg2, y2, dis, b2p)


# speedup vs baseline: 50.6007x; 1.0066x over previous
import functools

import jax
import jax.numpy as jnp
from jax import lax
from jax.experimental import pallas as pl
from jax.experimental.pallas import tpu as pltpu
from jax.experimental.pallas import tpu_sc as plsc

N = 10000
E = 320000
DF = 128
DH = 16
NCLS = 10
NC, NS, L = 2, 16, 16
NW = NC * NS
NPAD = 10240
RPT = NPAD // NS
CHUNK = 128
EPW = E // NW
NCH = 80
EPWP = NCH * CHUNK
NBUF = 8
NROUND = NCH // NBUF

_mesh = plsc.VectorSubcoreMesh(core_axis_name="c", subcore_axis_name="s")
_sc_params = pltpu.CompilerParams(use_tc_tiling_on_sc=False)


def _bcast_lane(vec, j):
    idx = jnp.full((L, 1), j, dtype=jnp.int32)
    dnums = lax.GatherDimensionNumbers(
        offset_dims=(), collapsed_slice_dims=(0,), start_index_map=(0,))
    return lax.gather(vec, idx, dnums, (1,),
                      mode=lax.GatherScatterMode.PROMISE_IN_BOUNDS)


def _zero_buf(buf):
    @pl.loop(0, CHUNK)
    def _(i):
        buf.at[i][...] = jnp.zeros((L,), jnp.float32)


def _export_slice(sh, out_hbm, c, s, buf):
    for t in range(RPT // CHUNK):
        off = s * RPT + t * CHUNK
        pltpu.sync_copy(sh.at[pl.ds(off, CHUNK)], buf)
        pltpu.sync_copy(buf, out_hbm.at[pl.ds(c * NPAD + off, CHUNK)])


@functools.partial(
    pl.kernel,
    out_type=jax.ShapeDtypeStruct((NC * NPAD, DH), jnp.float32),
    mesh=_mesh,
    scratch_types=[
        pltpu.VMEM((NCH, CHUNK), jnp.int32),
        pltpu.VMEM((NCH, CHUNK), jnp.float32),
    ] + [pltpu.VMEM((CHUNK, DH), jnp.float32)] * NBUF + [
        pltpu.VMEM_SHARED((NPAD, DH), jnp.float32),
    ] + [pltpu.SemaphoreType.DMA] * NBUF,
    compiler_params=_sc_params,
)
def _deg_kernel(col_hbm, ew_hbm, out_hbm, col_v, ew_v, *rest):
    sbufs = rest[:NBUF]
    deg_sh = rest[NBUF]
    ssems = rest[NBUF + 1:]
    c = lax.axis_index("c")
    s = lax.axis_index("s")
    wid = c * NS + s

    _zero_buf(sbufs[0])
    for t in range(RPT // CHUNK):
        pltpu.sync_copy(sbufs[0], deg_sh.at[pl.ds(s * RPT + t * CHUNK, CHUNK)])
    pltpu.sync_copy(col_hbm.at[wid], col_v)
    pltpu.sync_copy(ew_hbm.at[wid], ew_v)
    plsc.subcore_barrier()

    @pl.loop(0, NROUND)
    def _(m):
        for b in range(NBUF):
            k = m * NBUF + b

            @pl.when(m > 0)
            def _():
                pltpu.make_async_copy(
                    sbufs[b], deg_sh.at[col_v.at[k - NBUF]], ssems[b]).wait()

            @pl.loop(0, CHUNK, step=L)
            def _(g):
                ew16 = ew_v.at[k][pl.ds(g, L)]
                for j in range(L):
                    sbufs[b].at[g + j][...] = (
                        sbufs[b].at[g + j][...] * 0.0 + _bcast_lane(ew16, j))

            pltpu.async_copy(sbufs[b], deg_sh.at[col_v.at[k]], ssems[b],
                             add=True)

    for b in range(NBUF):
        pltpu.make_async_copy(
            sbufs[b], deg_sh.at[col_v.at[NCH - NBUF + b]], ssems[b]).wait()
    plsc.subcore_barrier()
    _export_slice(deg_sh, out_hbm, c, s, sbufs[0])


@functools.partial(
    pl.kernel,
    out_type=jax.ShapeDtypeStruct((NC * NPAD, DH), jnp.float32),
    mesh=_mesh,
    scratch_types=[
        pltpu.VMEM((NCH, CHUNK), jnp.int32),
        pltpu.VMEM((NCH, CHUNK), jnp.int32),
        pltpu.VMEM((NCH, CHUNK), jnp.float32),
    ] + [pltpu.VMEM((CHUNK, DH), jnp.float32)] * (2 * NBUF) + [
        pltpu.VMEM_SHARED((NPAD, DH), jnp.float32),
        pltpu.VMEM_SHARED((NPAD, DH), jnp.float32),
    ] + [pltpu.SemaphoreType.DMA] * (2 * NBUF),
    compiler_params=_sc_params,
)
def _msg_kernel(y_hbm, row_hbm, col_hbm, ew_hbm, out_hbm,
                row_v, col_v, ew_v, *rest):
    gbufs = rest[:NBUF]
    sbufs = rest[NBUF:2 * NBUF]
    agg_sh = rest[2 * NBUF]
    y_sh = rest[2 * NBUF + 1]
    gsems = rest[2 * NBUF + 2:3 * NBUF + 2]
    ssems = rest[3 * NBUF + 2:]
    c = lax.axis_index("c")
    s = lax.axis_index("s")
    wid = c * NS + s

    for t in range(RPT // CHUNK):
        sl = pl.ds(s * RPT + t * CHUNK, CHUNK)
        pltpu.sync_copy(y_hbm.at[sl], sbufs[0])
        pltpu.sync_copy(sbufs[0], y_sh.at[sl])
    _zero_buf(sbufs[0])
    for t in range(RPT // CHUNK):
        pltpu.sync_copy(sbufs[0], agg_sh.at[pl.ds(s * RPT + t * CHUNK, CHUNK)])
    pltpu.sync_copy(row_hbm.at[wid], row_v)
    pltpu.sync_copy(col_hbm.at[wid], col_v)
    pltpu.sync_copy(ew_hbm.at[wid], ew_v)
    plsc.subcore_barrier()

    for b in range(NBUF):
        pltpu.async_copy(y_sh.at[row_v.at[b]], gbufs[b], gsems[b])

    @pl.loop(0, NROUND)
    def _(m):
        for b in range(NBUF):
            k = m * NBUF + b

            @pl.when(m > 0)
            def _():
                pltpu.make_async_copy(
                    sbufs[b], agg_sh.at[col_v.at[k - NBUF]], ssems[b]).wait()

            pltpu.make_async_copy(
                y_sh.at[row_v.at[k]], gbufs[b], gsems[b]).wait()

            @pl.loop(0, CHUNK, step=L)
            def _(g):
                ew16 = ew_v.at[k][pl.ds(g, L)]
                for j in range(L):
                    w = _bcast_lane(ew16, j)
                    sbufs[b].at[g + j][...] = gbufs[b].at[g + j][...] * w

            @pl.when(m < NROUND - 1)
            def _():
                pltpu.async_copy(
                    y_sh.at[row_v.at[k + NBUF]], gbufs[b], gsems[b])

            pltpu.async_copy(sbufs[b], agg_sh.at[col_v.at[k]], ssems[b],
                             add=True)

    for b in range(NBUF):
        pltpu.make_async_copy(
            sbufs[b], agg_sh.at[col_v.at[NCH - NBUF + b]], ssems[b]).wait()
    plsc.subcore_barrier()
    _export_slice(agg_sh, out_hbm, c, s, sbufs[0])


def _tc2_body(deg_ref, x_ref, w_ref, y_ref, dis_ref):
    deg = deg_ref[:NPAD] + deg_ref[NPAD:]
    dis = lax.rsqrt(deg[:, 0:1] + 1.0)
    dis_ref[...] = dis
    xw = jnp.dot(x_ref[...], w_ref[...], preferred_element_type=jnp.float32)
    y_ref[:N, :] = xw * dis[:N]
    y_ref[N:, :] = jnp.zeros((NPAD - N, DH), jnp.float32)


_tc2 = pl.pallas_call(
    _tc2_body,
    out_shape=(jax.ShapeDtypeStruct((NPAD, DH), jnp.float32),
               jax.ShapeDtypeStruct((NPAD, 1), jnp.float32)))


def _tc3_body(agg_ref, y_ref, dis_ref, b1_ref, w2_ref, y2_ref):
    z = agg_ref[:NPAD] + agg_ref[NPAD:] + y_ref[...]
    h = jnp.maximum(z * dis_ref[...] + b1_ref[...], 0.0)
    xw2 = jnp.dot(h, w2_ref[...], preferred_element_type=jnp.float32)
    y2_ref[...] = xw2 * dis_ref[...]


_tc3 = pl.pallas_call(
    _tc3_body, out_shape=jax.ShapeDtypeStruct((NPAD, DH), jnp.float32))


def _tc4_body(agg_ref, y2_ref, dis_ref, b2_ref, o_ref):
    z = agg_ref[:NPAD] + agg_ref[NPAD:] + y2_ref[...]
    logits = (z * dis_ref[...] + b2_ref[...])[:N, :NCLS]
    m = jnp.max(logits, axis=1, keepdims=True)
    lse = jnp.log(jnp.sum(jnp.exp(logits - m), axis=1, keepdims=True)) + m
    o_ref[...] = logits - lse


_tc4 = pl.pallas_call(
    _tc4_body, out_shape=jax.ShapeDtypeStruct((N, NCLS), jnp.float32))


def _pad_edges(a, fill):
    a = a.reshape(NW, EPW)
    pad = jnp.full((NW, EPWP - EPW), fill, dtype=a.dtype)
    return jnp.concatenate([a, pad], axis=1).reshape(NW, NCH, CHUNK)


def kernel(x, edge_index, edge_weight, W1, b1, W2, b2):
    rowp = _pad_edges(edge_index[0].astype(jnp.int32), 0)
    colp = _pad_edges(edge_index[1].astype(jnp.int32), 0)
    ewp = _pad_edges(edge_weight, 0.0)
    W2p = jnp.pad(W2, ((0, 0), (0, DH - NCLS)))
    b2p = jnp.pad(b2, (0, DH - NCLS))

    deg2 = _deg_kernel(colp, ewp)
    y1, dis = _tc2(deg2, x, W1)
    agg1 = _msg_kernel(y1, rowp, colp, ewp)
    y2 = _tc3(agg1, y1, dis, b1, W2p)
    agg2 = _msg_kernel(y2, rowp, colp, ewp)
    return _tc4(agg2, y2, dis, b2p)
